# baseline (speedup 1.0000x reference)
import jax
import jax.numpy as jnp
from jax import lax
from jax.experimental import pallas as pl
from jax.experimental.pallas import tpu as pltpu
from jax.experimental.pallas import tpu_sc as plsc

N = 10000
E = 160000
B = 128
NBLK = 1280
E_PAD = NBLK * B
NROW = 10112
RPT = NROW // 16

GB = 8
NBLK_A = 1408
E_PAD_A = NBLK_A * B

SPAN = 2560
NACC = 2688
NACC2 = 2 * NACC
NPT = NACC2 // 16
BN = 320

_mesh = plsc.VectorSubcoreMesh(core_axis_name="c", subcore_axis_name="s")


def _sc_degree(dst_blocks, ones, zeros):
    def body(dst_hbm, ones_hbm, z_hbm, out_hbm, dst_v, ones_v, accum):
        c = lax.axis_index("c")
        s = lax.axis_index("s")
        pltpu.sync_copy(ones_hbm, ones_v)
        pltpu.sync_copy(z_hbm, accum.at[pl.ds(s * RPT, RPT)])
        plsc.subcore_barrier()

        half = NBLK // 2
        bpt = half // 16

        def grp(g, carry):
            base = c * half + s * bpt + g * 8
            pltpu.sync_copy(dst_hbm.at[pl.ds(base, 8)], dst_v)
            for i in range(8):
                pltpu.sync_copy(ones_v, accum.at[dst_v.at[i]], add=True)
            return carry

        lax.fori_loop(0, bpt // 8, grp, 0)
        plsc.subcore_barrier()
        pltpu.sync_copy(accum.at[pl.ds(s * RPT, RPT)],
                        out_hbm.at[c].at[pl.ds(s * RPT, RPT)])

    return pl.kernel(
        body,
        out_type=jax.ShapeDtypeStruct((2, NROW, 128), jnp.float32),
        mesh=_mesh,
        scratch_types=[
            pltpu.VMEM((8, B), jnp.int32),
            pltpu.VMEM((B, 128), jnp.float32),
            pltpu.VMEM_SHARED((NROW, 128), jnp.float32),
        ],
    )(dst_blocks, ones, zeros)


def _sc_aggregate(src_blk, dst_blk, params, table, zeros):
    def body(src_hbm, dst_hbm, par_hbm, tab_hbm, z_hbm, out_hbm,
             src_v, dst_v, par_v, rows0, rows1, accum,
             gsem0, gsem1, ssem0, ssem1):
        c = lax.axis_index("c")
        s = lax.axis_index("s")
        pltpu.sync_copy(par_hbm, par_v)

        bufs = (rows0, rows1)
        gsems = (gsem0, gsem1)
        ssems = (ssem0, ssem1)

        for p in range(4):
            @pl.when(c == (p % 2))
            def _(p=p):
                pltpu.sync_copy(z_hbm, accum.at[pl.ds(s * NPT, NPT)])
                plsc.subcore_barrier()

                prow = par_v[p, pl.ds(0, 16)]
                blo = prow[0]
                tpb = prow[1]

                def group(g, carry):
                    base = pl.multiple_of(blo + s * tpb + g * GB, 8)
                    pltpu.sync_copy(src_hbm.at[pl.ds(base, GB)], src_v)
                    pltpu.sync_copy(dst_hbm.at[p].at[pl.ds(base, GB)], dst_v)
                    halves = tuple(
                        (b.reshape(2 * B, 128).at[pl.ds(0, B)],
                         b.reshape(2 * B, 128).at[pl.ds(B, B)]) for b in bufs)

                    gd = [None] * GB
                    sd = [None] * GB
                    gd[0] = pltpu.async_copy(
                        tab_hbm.at[src_v.at[0]], rows0, gsems[0])
                    for i in range(GB):
                        if i + 1 < GB:
                            if i >= 1:
                                for d in sd[i - 1]:
                                    d.wait()
                            gd[i + 1] = pltpu.async_copy(
                                tab_hbm.at[src_v.at[i + 1]],
                                bufs[(i + 1) % 2], gsems[(i + 1) % 2])
                        gd[i].wait()
                        ha, hb = halves[i % 2]
                        sd[i] = (
                            pltpu.async_copy(ha, accum.at[dst_v.at[i].at[0]],
                                             ssems[i % 2], add=True),
                            pltpu.async_copy(hb, accum.at[dst_v.at[i].at[1]],
                                             ssems[i % 2], add=True),
                        )
                    for d in sd[GB - 2] + sd[GB - 1]:
                        d.wait()
                    return carry

                lax.fori_loop(0, tpb // GB, group, 0)

                plsc.subcore_barrier()
                pltpu.sync_copy(accum.at[pl.ds(s * NPT, NPT)],
                                out_hbm.at[p].at[pl.ds(s * NPT, NPT)])
                plsc.subcore_barrier()

    return pl.kernel(
        body,
        out_type=jax.ShapeDtypeStruct((4, NACC2, 128), jnp.float32),
        mesh=_mesh,
        scratch_types=[
            pltpu.VMEM((GB, B), jnp.int32),
            pltpu.VMEM((GB, 2, B), jnp.int32),
            pltpu.VMEM((8, 128), jnp.int32),
            pltpu.VMEM((B, 2, 128), jnp.float32),
            pltpu.VMEM((B, 2, 128), jnp.float32),
            pltpu.VMEM_SHARED((NACC2, 128), jnp.float32),
            pltpu.SemaphoreType.DMA,
            pltpu.SemaphoreType.DMA,
            pltpu.SemaphoreType.DMA,
            pltpu.SemaphoreType.DMA,
        ],
    )(src_blk, dst_blk, params, table, zeros)


def _window_params(off):
    rows = []
    for p in range(4):
        blo = (off[p] // (8 * B)) * 8
        bhi = -(-off[p + 1] // (8 * B)) * 8
        cdiv16 = -(-(bhi - blo) // 16)
        tpb = -(-cdiv16 // GB) * GB
        rows.append(jnp.stack([blo, tpb]))
    par = jnp.zeros((8, 128), jnp.int32)
    return par.at[:4, :2].set(jnp.stack(rows))


def _elu(z):
    return jnp.where(z > 0, z, jnp.exp(jnp.minimum(z, 0.0)) - 1.0)


def _inv_deg(deg_ref):
    deg = deg_ref[0, :, 0:1] + deg_ref[1, :, 0:1]
    return 1.0 / jnp.maximum(deg, 1.0)


def _d0_body(agg_ref, x_ref, deg_ref, wl_ref, wr_ref, bl_ref, h1a_ref, h1b_ref):
    inv = _inv_deg(deg_ref)
    acc = jnp.broadcast_to(bl_ref[...], (BN, 512))
    acc = acc + jnp.dot(_agg_rows(agg_ref) * inv, wl_ref[...],
                        preferred_element_type=jnp.float32)
    acc = acc + jnp.dot(x_ref[...], wr_ref[...],
                        preferred_element_type=jnp.float32)
    h = _elu(acc)
    h1a_ref[...] = h[:, :256]
    h1b_ref[...] = h[:, 256:]


def _d1_body(agga_ref, aggb_ref, ha_ref, hb_ref, deg_ref, wla_ref, wlb_ref,
             wra_ref, wrb_ref, bl_ref, wl2_ref, h2_ref, p2_ref):
    inv = _inv_deg(deg_ref)
    acc = jnp.broadcast_to(bl_ref[...], (BN, 512))
    acc = acc + jnp.dot(_agg_rows(agga_ref) * inv, wla_ref[...],
                        preferred_element_type=jnp.float32)
    acc = acc + jnp.dot(_agg_rows(aggb_ref) * inv, wlb_ref[...],
                        preferred_element_type=jnp.float32)
    acc = acc + jnp.dot(ha_ref[...], wra_ref[...],
                        preferred_element_type=jnp.float32)
    acc = acc + jnp.dot(hb_ref[...], wrb_ref[...],
                        preferred_element_type=jnp.float32)
    h2 = _elu(acc)
    h2_ref[...] = h2
    p2_ref[...] = jnp.dot(h2, wl2_ref[...], preferred_element_type=jnp.float32)


def _d2_body(agg_ref, h_ref, deg_ref, wr_ref, bl_ref, out_ref):
    inv = _inv_deg(deg_ref)
    z = _agg_rows(agg_ref) * inv + jnp.broadcast_to(bl_ref[...], (BN, 256))
    z = z + jnp.dot(h_ref[...], wr_ref[...], preferred_element_type=jnp.float32)
    m = jnp.max(z, axis=1, keepdims=True)
    ez = jnp.exp(z - m)
    lse = jnp.log(jnp.sum(ez, axis=1, keepdims=True))
    out_ref[...] = z - m - lse


def _agg_spec():
    return pl.BlockSpec((1, 2 * BN, 128), lambda i: (i // 8, i % 8, 0))


def _agg_rows(ref):
    return ref[0].reshape(BN, 256)


def _row_spec(w):
    return pl.BlockSpec((BN, w), lambda i: (i, 0))


def _deg_spec():
    return pl.BlockSpec((2, BN, 128), lambda i: (0, i, 0))


def _full_spec(shape):
    nz = len(shape) * (0,)
    return pl.BlockSpec(shape, lambda i, nz=nz: nz)


def kernel(x, edge_index, Wl0, bl0, Wr0, Wl1, bl1, Wr1, Wl2, bl2, Wr2):
    f32 = jnp.float32
    src = edge_index[0].astype(jnp.int32)
    dst = edge_index[1].astype(jnp.int32)

    order = jnp.argsort(dst)
    src_s = src[order]
    dst_s = dst[order]
    npad = E_PAD_A - E
    big = jnp.int32(1 << 20)
    src_sp = jnp.concatenate([src_s, jnp.zeros((npad,), jnp.int32)])
    dst_sp = jnp.concatenate([dst_s, jnp.full((npad,), big, jnp.int32)])
    off = jnp.searchsorted(
        dst_s, jnp.arange(0, 4 * SPAN + 1, SPAN, dtype=jnp.int32)
    ).astype(jnp.int32)
    par = _window_params(off)
    src_blk = src_sp.reshape(NBLK_A, B)
    dump = SPAN + (jnp.arange(E_PAD_A, dtype=jnp.int32) % 128)
    lows = (jnp.arange(4, dtype=jnp.int32) * SPAN)[:, None]
    rel = dst_sp[None, :] - lows
    d = jnp.where((rel >= 0) & (rel < SPAN), rel, dump[None, :])
    dst_blk = (2 * d[:, :, None] +
               jnp.arange(2, dtype=jnp.int32)).reshape(4, NBLK_A, 2, B)

    dnpad = E_PAD - E
    dst_deg = jnp.concatenate(
        [dst, N + (jnp.arange(dnpad, dtype=jnp.int32) % (NROW - N))]
    ).reshape(NBLK, B)

    zeros_a = jnp.zeros((NPT, 128), f32)
    zeros_d = jnp.zeros((RPT, 128), f32)
    ones = jnp.ones((B, 128), f32)

    bl0r = bl0.reshape(1, 512)
    bl1r = bl1.reshape(1, 512)
    bl2r = bl2.reshape(1, 256)

    degp = _sc_degree(dst_deg, ones, zeros_d)

    agg0 = _sc_aggregate(src_blk, dst_blk, par, x.reshape(N, 2, 128), zeros_a)
    h1a, h1b = pl.pallas_call(
        _d0_body,
        grid=(32,),
        in_specs=[_agg_spec(), _row_spec(256), _deg_spec(),
                  _full_spec((256, 512)), _full_spec((256, 512)),
                  _full_spec((1, 512))],
        out_specs=[_row_spec(256), _row_spec(256)],
        out_shape=[jax.ShapeDtypeStruct((N, 256), f32),
                   jax.ShapeDtypeStruct((N, 256), f32)],
    )(agg0, x, degp, Wl0, Wr0, bl0r)

    agg1a = _sc_aggregate(src_blk, dst_blk, par, h1a.reshape(N, 2, 128), zeros_a)
    agg1b = _sc_aggregate(src_blk, dst_blk, par, h1b.reshape(N, 2, 128), zeros_a)
    h2, p2 = pl.pallas_call(
        _d1_body,
        grid=(32,),
        in_specs=[_agg_spec(), _agg_spec(), _row_spec(256), _row_spec(256),
                  _deg_spec(),
                  _full_spec((256, 512)), _full_spec((256, 512)),
                  _full_spec((256, 512)), _full_spec((256, 512)),
                  _full_spec((1, 512)), _full_spec((512, 256))],
        out_specs=[_row_spec(512), _row_spec(256)],
        out_shape=[jax.ShapeDtypeStruct((N, 512), f32),
                   jax.ShapeDtypeStruct((N, 256), f32)],
    )(agg1a, agg1b, h1a, h1b, degp,
      Wl1[:256], Wl1[256:], Wr1[:256], Wr1[256:], bl1r, Wl2)

    agg2 = _sc_aggregate(src_blk, dst_blk, par, p2.reshape(N, 2, 128), zeros_a)
    out = pl.pallas_call(
        _d2_body,
        grid=(32,),
        in_specs=[_agg_spec(), _row_spec(512), _deg_spec(),
                  _full_spec((512, 256)), _full_spec((1, 256))],
        out_specs=_row_spec(256),
        out_shape=jax.ShapeDtypeStruct((N, 256), f32),
    )(agg2, h2, degp, Wr2, bl2r)
    return out


# speedup vs baseline: 1.9778x; 1.9778x over previous
import functools

import jax
import jax.numpy as jnp
from jax import lax
from jax.experimental import pallas as pl
from jax.experimental.pallas import tpu as pltpu
from jax.experimental.pallas import tpu_sc as plsc

N = 10000
E = 160000
B = 128
NBLK = 1280
E_PAD = NBLK * B
BPT = NBLK // 16
NROW = 10112
RPT = NROW // 16

_mesh = plsc.VectorSubcoreMesh(core_axis_name="c", subcore_axis_name="s")


def _sc_degree(dst_blocks, ones, zeros):
    def body(dst_hbm, ones_hbm, z_hbm, out_hbm, dst_v, ones_v, accum):
        c = lax.axis_index("c")
        s = lax.axis_index("s")
        pltpu.sync_copy(ones_hbm, ones_v)
        pltpu.sync_copy(z_hbm, accum.at[pl.ds(s * RPT, RPT)])
        plsc.subcore_barrier()

        half = NBLK // 2
        bpt = half // 16

        def grp(g, carry):
            base = c * half + s * bpt + g * 8
            pltpu.sync_copy(dst_hbm.at[pl.ds(base, 8)], dst_v)
            for i in range(8):
                pltpu.sync_copy(ones_v, accum.at[dst_v.at[i]], add=True)
            return carry

        lax.fori_loop(0, bpt // 8, grp, 0)
        plsc.subcore_barrier()
        pltpu.sync_copy(accum.at[pl.ds(s * RPT, RPT)],
                        out_hbm.at[c].at[pl.ds(s * RPT, RPT)])

    return pl.kernel(
        body,
        out_type=jax.ShapeDtypeStruct((2, NROW, 128), jnp.float32),
        mesh=_mesh,
        scratch_types=[
            pltpu.VMEM((8, B), jnp.int32),
            pltpu.VMEM((B, 128), jnp.float32),
            pltpu.VMEM_SHARED((NROW, 128), jnp.float32),
        ],
    )(dst_blocks, ones, zeros)


def _sc_aggregate(nchunks, srcs_flat, dst_blocks, table, zeros):
    GB = 8

    def body(src_hbm, dst_hbm, tab_hbm, z_hbm, out_hbm,
             src_v, dst_v, rows0, rows1, accum, gsem0, gsem1, ssem0, ssem1):
        c = lax.axis_index("c")
        s = lax.axis_index("s")

        for j in range(nchunks):
            @pl.when(c == (j % 2))
            def _(j=j):
                pltpu.sync_copy(z_hbm, accum.at[pl.ds(s * RPT, RPT)])
                plsc.subcore_barrier()

                bufs = (rows0, rows1)
                gsems = (gsem0, gsem1)
                ssems = (ssem0, ssem1)

                def group(g, carry):
                    base = s * BPT + g * GB
                    pltpu.sync_copy(src_hbm.at[pl.ds(j * NBLK + base, GB)], src_v)
                    pltpu.sync_copy(dst_hbm.at[pl.ds(base, GB)], dst_v)
                    pend = pltpu.async_copy(
                        tab_hbm.at[src_v.at[0]], rows0, gsem0)
                    for i in range(1, GB):
                        nxt = pltpu.async_copy(
                            tab_hbm.at[src_v.at[i]], bufs[i % 2],
                            gsems[i % 2])
                        pend.wait()
                        pltpu.sync_copy(bufs[(i - 1) % 2],
                                        accum.at[dst_v.at[i - 1]], add=True)
                        pend = nxt
                    pend.wait()
                    pltpu.sync_copy(bufs[(GB - 1) % 2],
                                    accum.at[dst_v.at[GB - 1]], add=True)
                    return carry

                lax.fori_loop(0, BPT // GB, group, 0)

                plsc.subcore_barrier()
                pltpu.sync_copy(accum.at[pl.ds(s * RPT, RPT)],
                                out_hbm.at[pl.ds(j * NROW + s * RPT, RPT)])
                plsc.subcore_barrier()

    return pl.kernel(
        body,
        out_type=jax.ShapeDtypeStruct((nchunks * NROW, 128), jnp.float32),
        mesh=_mesh,
        scratch_types=[
            pltpu.VMEM((8, B), jnp.int32),
            pltpu.VMEM((8, B), jnp.int32),
            pltpu.VMEM((B, 128), jnp.float32),
            pltpu.VMEM((B, 128), jnp.float32),
            pltpu.VMEM_SHARED((NROW, 128), jnp.float32),
            pltpu.SemaphoreType.DMA,
            pltpu.SemaphoreType.DMA,
            pltpu.SemaphoreType.DMA,
            pltpu.SemaphoreType.DMA,
        ],
    )(srcs_flat, dst_blocks, table, zeros)


BN = 1000


def _elu(z):
    return jnp.where(z > 0, z, jnp.exp(jnp.minimum(z, 0.0)) - 1.0)


def _inv_deg(deg_ref):
    deg = deg_ref[0, :, 0:1] + deg_ref[1, :, 0:1]
    return 1.0 / jnp.maximum(deg, 1.0)


def _d0_body(agg_ref, x_ref, deg_ref, wl_ref, wr_ref, bl_ref, out_ref):
    inv = _inv_deg(deg_ref)
    acc = jnp.broadcast_to(bl_ref[...], (BN, 512))
    for ci in range(2):
        acc = acc + jnp.dot(agg_ref[ci] * inv, wl_ref[ci],
                            preferred_element_type=jnp.float32)
        acc = acc + jnp.dot(x_ref[ci], wr_ref[ci],
                            preferred_element_type=jnp.float32)
    h = _elu(acc)
    for co in range(4):
        out_ref[co] = h[:, co * 128:(co + 1) * 128]


def _d1_body(agg_ref, h_ref, deg_ref, wl_ref, wr_ref, bl_ref, wl2_ref,
             h2_ref, p2_ref):
    inv = _inv_deg(deg_ref)
    acc = jnp.broadcast_to(bl_ref[...], (BN, 512))
    for ci in range(4):
        acc = acc + jnp.dot(agg_ref[ci] * inv, wl_ref[ci],
                            preferred_element_type=jnp.float32)
        acc = acc + jnp.dot(h_ref[ci], wr_ref[ci],
                            preferred_element_type=jnp.float32)
    h2 = _elu(acc)
    p2 = jnp.zeros((BN, 256), jnp.float32)
    for ci in range(4):
        hc = h2[:, ci * 128:(ci + 1) * 128]
        h2_ref[ci] = hc
        p2 = p2 + jnp.dot(hc, wl2_ref[ci], preferred_element_type=jnp.float32)
    for co in range(2):
        p2_ref[co] = p2[:, co * 128:(co + 1) * 128]


def _d2_body(agg_ref, h_ref, deg_ref, wr_ref, bl_ref, out_ref):
    inv = _inv_deg(deg_ref)
    z = jnp.concatenate([agg_ref[0] * inv, agg_ref[1] * inv], axis=1)
    z = z + jnp.broadcast_to(bl_ref[...], (BN, 256))
    for ci in range(4):
        z = z + jnp.dot(h_ref[ci], wr_ref[ci], preferred_element_type=jnp.float32)
    m = jnp.max(z, axis=1, keepdims=True)
    ez = jnp.exp(z - m)
    lse = jnp.log(jnp.sum(ez, axis=1, keepdims=True))
    out_ref[...] = z - m - lse


def _node_spec(cdim, fdim):
    return pl.BlockSpec((cdim, BN, fdim), lambda i: (0, i, 0))


def _full_spec(shape):
    nz = len(shape) * (0,)
    return pl.BlockSpec(shape, lambda i, nz=nz: nz)


def kernel(x, edge_index, Wl0, bl0, Wr0, Wl1, bl1, Wr1, Wl2, bl2, Wr2):
    f32 = jnp.float32
    src = edge_index[0].astype(jnp.int32)
    dst = edge_index[1].astype(jnp.int32)
    npad = E_PAD - E
    src_p = jnp.concatenate([src, jnp.zeros((npad,), jnp.int32)])
    dst_p = jnp.concatenate(
        [dst, N + (jnp.arange(npad, dtype=jnp.int32) % (NROW - N))])
    dst_blocks = dst_p.reshape(NBLK, B)

    def chunk_srcs(nchunks):
        return (src_p[None, :] +
                (jnp.arange(nchunks, dtype=jnp.int32) * N)[:, None]).reshape(
                    nchunks * NBLK, B)

    srcs2 = chunk_srcs(2)
    srcs4 = chunk_srcs(4)

    zeros = jnp.zeros((RPT, 128), f32)
    ones = jnp.ones((B, 128), f32)

    xc = x.reshape(N, 2, 128).transpose(1, 0, 2)
    wl0 = Wl0.reshape(2, 128, 512)
    wr0 = Wr0.reshape(2, 128, 512)
    wl1 = Wl1.reshape(4, 128, 512)
    wr1 = Wr1.reshape(4, 128, 512)
    wl2 = Wl2.reshape(4, 128, 256)
    wr2 = Wr2.reshape(4, 128, 256)
    bl0r = bl0.reshape(1, 512)
    bl1r = bl1.reshape(1, 512)
    bl2r = bl2.reshape(1, 256)

    degp = _sc_degree(dst_blocks, ones, zeros)

    agg0 = _sc_aggregate(2, srcs2, dst_blocks, xc.reshape(2 * N, 128), zeros)
    agg0 = agg0.reshape(2, NROW, 128)
    h1 = pl.pallas_call(
        _d0_body,
        grid=(N // BN,),
        in_specs=[_node_spec(2, 128), _node_spec(2, 128), _node_spec(2, 128),
                  _full_spec((2, 128, 512)), _full_spec((2, 128, 512)),
                  _full_spec((1, 512))],
        out_specs=_node_spec(4, 128),
        out_shape=jax.ShapeDtypeStruct((4, N, 128), f32),
    )(agg0, xc, degp, wl0, wr0, bl0r)

    agg1 = _sc_aggregate(4, srcs4, dst_blocks, h1.reshape(4 * N, 128), zeros)
    agg1 = agg1.reshape(4, NROW, 128)
    h2, p2 = pl.pallas_call(
        _d1_body,
        grid=(N // BN,),
        in_specs=[_node_spec(4, 128), _node_spec(4, 128), _node_spec(2, 128),
                  _full_spec((4, 128, 512)), _full_spec((4, 128, 512)),
                  _full_spec((1, 512)), _full_spec((4, 128, 256))],
        out_specs=[_node_spec(4, 128), _node_spec(2, 128)],
        out_shape=[jax.ShapeDtypeStruct((4, N, 128), f32),
                   jax.ShapeDtypeStruct((2, N, 128), f32)],
    )(agg1, h1, degp, wl1, wr1, bl1r, wl2)

    agg2 = _sc_aggregate(2, srcs2, dst_blocks, p2.reshape(2 * N, 128), zeros)
    agg2 = agg2.reshape(2, NROW, 128)
    out = pl.pallas_call(
        _d2_body,
        grid=(N // BN,),
        in_specs=[_node_spec(2, 128), _node_spec(4, 128), _node_spec(2, 128),
                  _full_spec((4, 128, 256)), _full_spec((1, 256))],
        out_specs=pl.BlockSpec((BN, 256), lambda i: (i, 0)),
        out_shape=jax.ShapeDtypeStruct((N, 256), f32),
    )(agg2, h2, degp, wr2, bl2r)
    return out
